# Initial kernel scaffold; baseline (speedup 1.0000x reference)
import jax
import jax.numpy as jnp
from jax.experimental import pallas as pl


def kernel(x_t, t, emb, temb, W, b):
    raise NotImplementedError("write your pallas kernel here")


# speedup vs baseline: 21.1856x; 21.1856x over previous
import numpy as np
import jax
import jax.numpy as jnp
from jax.experimental import pallas as pl
from jax.experimental.pallas import tpu as pltpu

_B, _L, _K, _D, _T = 128, 8192, 4, 256, 100
_TL = 8192
_NC = _TL // 2

_KS0 = np.uint32(0)
_KS1 = np.uint32(42)
_KS2 = np.uint32(0x1BD11BDA) ^ _KS0 ^ _KS1
_ROT = ((13, 15, 26, 6), (17, 29, 16, 24))
_TINY = np.float32(np.finfo(np.float32).tiny)


def _rotl(x, r):
    return (x << np.uint32(r)) | (x >> np.uint32(32 - r))


def _threefry_zero_hi(x1):
    ks = (_KS0, _KS1, _KS2)
    x0 = jnp.zeros_like(x1)
    x1 = x1 + ks[1]
    for i in range(5):
        for r in _ROT[i % 2]:
            x0 = x0 + x1
            x1 = _rotl(x1, r)
            x1 = x0 ^ x1
        x0 = x0 + ks[(i + 1) % 3]
        x1 = x1 + ks[(i + 2) % 3] + np.uint32(i + 1)
    return x0 ^ x1


def _sel4(v, a0, a1, a2, a3):
    return jnp.where(v == 0, a0, jnp.where(v == 1, a1, jnp.where(v == 2, a2, a3)))


def _ddpm_kernel(t_ref, x_ref, emb_ref, temb_ref, w_ref, b_ref, out_ref):
    i = pl.program_id(0)
    t0 = t_ref[i]

    h = emb_ref[...] + temb_ref[pl.ds(t0, 1), :]
    h = jax.nn.gelu(h)
    lg = jax.lax.dot_general(h, w_ref[...], (((1,), (0,)), ((), ())),
                             precision=jax.lax.Precision.HIGHEST,
                             preferred_element_type=jnp.float32)
    lg = lg + b_ref[...]
    m = jnp.maximum(jnp.maximum(lg[:, 0:1], lg[:, 1:2]),
                    jnp.maximum(lg[:, 2:3], lg[:, 3:4]))
    e = jnp.exp(lg - m)
    p = e / (e[:, 0:1] + e[:, 1:2] + e[:, 2:3] + e[:, 3:4])
    lp = jnp.log(p + np.float32(1e-20))

    gbest = p[:, 0:1]
    gidx = jnp.zeros((_K, 1), jnp.int32)
    for c in range(1, _K):
        hit = p[:, c:c + 1] > gbest
        gidx = jnp.where(hit, c, gidx)
        gbest = jnp.where(hit, p[:, c:c + 1], gbest)

    base = i * (_L * _K)
    si = jax.lax.broadcasted_iota(jnp.int32, (8, _NC), 0)
    qi = jax.lax.broadcasted_iota(jnp.int32, (8, _NC), 1)
    ctr = (base + 4 * qi + (si & 3) + (si >> 2) * (2 * _TL)).astype(jnp.uint32)
    bits = _threefry_zero_hi(ctr)
    fb = jax.lax.bitcast_convert_type(
        (bits >> np.uint32(9)) | np.uint32(0x3F800000), jnp.float32)
    fb = fb - np.float32(1.0)
    u = jnp.maximum(_TINY, fb + _TINY)
    g = -jnp.log(-jnp.log(u))

    xv = x_ref[0]
    for half in range(2):
        v = xv[:, half * _NC:(half + 1) * _NC]
        r0 = 4 * half
        best = g[r0:r0 + 1, :] + _sel4(
            v, lp[0:1, 0:1], lp[1:2, 0:1], lp[2:3, 0:1], lp[3:4, 0:1])
        idx = jnp.zeros((1, _NC), jnp.int32)
        for c in range(1, _K):
            sc = g[r0 + c:r0 + c + 1, :] + _sel4(
                v, lp[0:1, c:c + 1], lp[1:2, c:c + 1], lp[2:3, c:c + 1],
                lp[3:4, c:c + 1])
            hit = sc > best
            idx = jnp.where(hit, c, idx)
            best = jnp.where(hit, sc, best)
        grd = _sel4(v, gidx[0:1], gidx[1:2], gidx[2:3], gidx[3:4])
        res = jnp.where(t0 == 0, grd, idx)
        out_ref[0, :, pl.ds(half * _NC, _NC)] = res


def kernel(x_t, t, emb, temb, W, b):
    x3 = x_t.reshape(_B, 1, _L)
    b2 = b.reshape(1, _K)
    out = pl.pallas_call(
        _ddpm_kernel,
        grid_spec=pltpu.PrefetchScalarGridSpec(
            num_scalar_prefetch=1,
            grid=(_B, _L // _TL),
            in_specs=[
                pl.BlockSpec((1, 1, _TL), lambda i, j, t_s: (i, 0, j)),
                pl.BlockSpec((_K, _D), lambda i, j, t_s: (0, 0)),
                pl.BlockSpec((_T, _D), lambda i, j, t_s: (0, 0)),
                pl.BlockSpec((_D, _K), lambda i, j, t_s: (0, 0)),
                pl.BlockSpec((1, _K), lambda i, j, t_s: (0, 0)),
            ],
            out_specs=pl.BlockSpec((1, 1, _TL), lambda i, j, t_s: (i, 0, j)),
        ),
        out_shape=jax.ShapeDtypeStruct((_B, 1, _L), jnp.int32),
        compiler_params=pltpu.CompilerParams(
            dimension_semantics=("parallel", "arbitrary")),
    )(t, x3, emb, temb, W, b2)
    return out.reshape(_B, _L)


# speedup vs baseline: 35.9575x; 1.6973x over previous
import numpy as np
import jax
import jax.numpy as jnp
from jax.experimental import pallas as pl
from jax.experimental.pallas import tpu as pltpu

_B, _L, _K, _D, _T = 128, 8192, 4, 256, 100
_TL = 8192
_NC = _TL // 2
_CW = 1024

_KS0 = np.uint32(0)
_KS1 = np.uint32(42)
_KS2 = np.uint32(0x1BD11BDA) ^ _KS0 ^ _KS1
_ROT = ((13, 15, 26, 6), (17, 29, 16, 24))
_TINY = np.float32(np.finfo(np.float32).tiny)


def _rotl(x, r):
    return (x << np.uint32(r)) | (x >> np.uint32(32 - r))


def _threefry_zero_hi(x1):
    ks = (_KS0, _KS1, _KS2)
    x0 = x1
    x1 = _rotl(x1, 13)
    x1 = x0 ^ x1
    for r in (15, 26, 6):
        x0 = x0 + x1
        x1 = _rotl(x1, r)
        x1 = x0 ^ x1
    x0 = x0 + ks[1]
    x1 = x1 + ks[2] + np.uint32(1)
    for i in range(1, 5):
        for r in _ROT[i % 2]:
            x0 = x0 + x1
            x1 = _rotl(x1, r)
            x1 = x0 ^ x1
        x0 = x0 + ks[(i + 1) % 3]
        x1 = x1 + ks[(i + 2) % 3] + np.uint32(i + 1)
    return x0 ^ x1


def _sel4(v, a0, a1, a2, a3):
    return jnp.where(v == 0, a0, jnp.where(v == 1, a1, jnp.where(v == 2, a2, a3)))


def _table_kernel(t2_ref, emb_ref, temb_ref, w_ref, b_ref, lp_ref, gr_ref):
    t2 = t2_ref[...]
    oh = (jax.lax.broadcasted_iota(jnp.int32, (_B, _T), 1) == t2)
    h_all = jax.lax.dot_general(oh.astype(jnp.float32), temb_ref[...],
                                (((1,), (0,)), ((), ())),
                                precision=jax.lax.Precision.HIGHEST,
                                preferred_element_type=jnp.float32)
    for v in range(_K):
        hv = jax.nn.gelu(h_all + emb_ref[v:v + 1, :])
        lg = jax.lax.dot_general(hv, w_ref[...], (((1,), (0,)), ((), ())),
                                 precision=jax.lax.Precision.HIGHEST,
                                 preferred_element_type=jnp.float32)
        lg = lg + b_ref[...]
        m = jnp.maximum(jnp.maximum(lg[:, 0:1], lg[:, 1:2]),
                        jnp.maximum(lg[:, 2:3], lg[:, 3:4]))
        e = jnp.exp(lg - m)
        p = e / (e[:, 0:1] + e[:, 1:2] + e[:, 2:3] + e[:, 3:4])
        lp_ref[:, 4 * v:4 * v + 4] = jnp.log(p + np.float32(1e-20))
        gbest = p[:, 0:1]
        gidx = jnp.zeros((_B, 1), jnp.int32)
        for c in range(1, _K):
            hit = p[:, c:c + 1] > gbest
            gidx = jnp.where(hit, c, gidx)
            gbest = jnp.where(hit, p[:, c:c + 1], gbest)
        gr_ref[:, v:v + 1] = gidx


def _ddpm_kernel(t_ref, x_ref, lp_ref, gr_ref, out_ref):
    i = pl.program_id(0)
    t0 = t_ref[i]

    def lp(v, c):
        return lp_ref[0, 0:1, 4 * v + c:4 * v + c + 1]

    def gr(v):
        return gr_ref[0, 0:1, v:v + 1]

    base = i * (_L * _K) + 42
    si = jax.lax.broadcasted_iota(jnp.int32, (8, _CW), 0)
    qi = jax.lax.broadcasted_iota(jnp.int32, (8, _CW), 1)
    pattern = (4 * qi + (si & 3) + (si >> 2) * (2 * _TL)).astype(jnp.uint32)

    xv = x_ref[0]
    for k in range(_NC // _CW):
        ctr = pattern + jnp.uint32(base + 4 * k * _CW)
        bits = _threefry_zero_hi(ctr)
        fb = jax.lax.bitcast_convert_type(
            (bits >> np.uint32(9)) | np.uint32(0x3F800000), jnp.float32)
        fb = fb - np.float32(1.0)
        u = jnp.maximum(_TINY, fb + _TINY)
        g = -jnp.log(-jnp.log(u))
        for half in range(2):
            off = half * _NC + k * _CW
            v = xv[:, off:off + _CW]
            r0 = 4 * half
            best = g[r0:r0 + 1, :] + _sel4(
                v, lp(0, 0), lp(1, 0), lp(2, 0), lp(3, 0))
            idx = jnp.zeros((1, _CW), jnp.int32)
            for c in range(1, _K):
                sc = g[r0 + c:r0 + c + 1, :] + _sel4(
                    v, lp(0, c), lp(1, c), lp(2, c), lp(3, c))
                hit = sc > best
                idx = jnp.where(hit, c, idx)
                best = jnp.where(hit, sc, best)
            grd = _sel4(v, gr(0), gr(1), gr(2), gr(3))
            res = jnp.where(t0 == 0, grd, idx)
            out_ref[0, :, pl.ds(off, _CW)] = res


def kernel(x_t, t, emb, temb, W, b):
    x3 = x_t.reshape(_B, 1, _L)
    b2 = b.reshape(1, _K)
    t2 = t.reshape(_B, 1)

    lp_all, gr_all = pl.pallas_call(
        _table_kernel,
        in_specs=[
            pl.BlockSpec((_B, 1), lambda: (0, 0)),
            pl.BlockSpec((_K, _D), lambda: (0, 0)),
            pl.BlockSpec((_T, _D), lambda: (0, 0)),
            pl.BlockSpec((_D, _K), lambda: (0, 0)),
            pl.BlockSpec((1, _K), lambda: (0, 0)),
        ],
        out_specs=[
            pl.BlockSpec((_B, 4 * _K), lambda: (0, 0)),
            pl.BlockSpec((_B, _K), lambda: (0, 0)),
        ],
        out_shape=[
            jax.ShapeDtypeStruct((_B, 4 * _K), jnp.float32),
            jax.ShapeDtypeStruct((_B, _K), jnp.int32),
        ],
    )(t2, emb, temb, W, b2)

    out = pl.pallas_call(
        _ddpm_kernel,
        grid_spec=pltpu.PrefetchScalarGridSpec(
            num_scalar_prefetch=1,
            grid=(_B, _L // _TL),
            in_specs=[
                pl.BlockSpec((1, 1, _TL), lambda i, j, t_s: (i, 0, j)),
                pl.BlockSpec((1, 1, 4 * _K), lambda i, j, t_s: (i, 0, 0)),
                pl.BlockSpec((1, 1, _K), lambda i, j, t_s: (i, 0, 0)),
            ],
            out_specs=pl.BlockSpec((1, 1, _TL), lambda i, j, t_s: (i, 0, j)),
        ),
        out_shape=jax.ShapeDtypeStruct((_B, 1, _L), jnp.int32),
        compiler_params=pltpu.CompilerParams(
            dimension_semantics=("parallel", "arbitrary")),
    )(t, x3, lp_all.reshape(_B, 1, 4 * _K), gr_all.reshape(_B, 1, _K))
    return out.reshape(_B, _L)


# speedup vs baseline: 38.2542x; 1.0639x over previous
import numpy as np
import jax
import jax.numpy as jnp
from jax.experimental import pallas as pl
from jax.experimental.pallas import tpu as pltpu

_B, _L, _K, _D, _T = 128, 8192, 4, 256, 100
_RP = 2
_TL = 8192
_NC = _TL // 2
_CW = 1024

_KS0 = np.uint32(0)
_KS1 = np.uint32(42)
_KS2 = np.uint32(0x1BD11BDA) ^ _KS0 ^ _KS1
_ROT = ((13, 15, 26, 6), (17, 29, 16, 24))
_TINY = np.float32(np.finfo(np.float32).tiny)


def _rotl(x, r):
    return (x << np.uint32(r)) | (x >> np.uint32(32 - r))


def _threefry_zero_hi(x1):
    ks = (_KS0, _KS1, _KS2)
    x0 = x1
    x1 = _rotl(x1, 13)
    x1 = x0 ^ x1
    for r in (15, 26, 6):
        x0 = x0 + x1
        x1 = _rotl(x1, r)
        x1 = x0 ^ x1
    x0 = x0 + ks[1]
    x1 = x1 + ks[2] + np.uint32(1)
    for i in range(1, 5):
        for r in _ROT[i % 2]:
            x0 = x0 + x1
            x1 = _rotl(x1, r)
            x1 = x0 ^ x1
        x0 = x0 + ks[(i + 1) % 3]
        x1 = x1 + ks[(i + 2) % 3] + np.uint32(i + 1)
    return x0 ^ x1


def _sel4(v, a0, a1, a2, a3):
    return jnp.where(v == 0, a0, jnp.where(v == 1, a1, jnp.where(v == 2, a2, a3)))


def _table_kernel(t2_ref, emb_ref, temb_ref, w_ref, b_ref, lp_ref, gr_ref):
    t2 = t2_ref[...]
    oh = (jax.lax.broadcasted_iota(jnp.int32, (_B, _T), 1) == t2)
    h_all = jax.lax.dot_general(oh.astype(jnp.float32), temb_ref[...],
                                (((1,), (0,)), ((), ())),
                                precision=jax.lax.Precision.HIGHEST,
                                preferred_element_type=jnp.float32)
    for v in range(_K):
        hv = jax.nn.gelu(h_all + emb_ref[v:v + 1, :])
        lg = jax.lax.dot_general(hv, w_ref[...], (((1,), (0,)), ((), ())),
                                 precision=jax.lax.Precision.HIGHEST,
                                 preferred_element_type=jnp.float32)
        lg = lg + b_ref[...]
        m = jnp.maximum(jnp.maximum(lg[:, 0:1], lg[:, 1:2]),
                        jnp.maximum(lg[:, 2:3], lg[:, 3:4]))
        e = jnp.exp(lg - m)
        p = e / (e[:, 0:1] + e[:, 1:2] + e[:, 2:3] + e[:, 3:4])
        lp_ref[:, 4 * v:4 * v + 4] = jnp.log(p + np.float32(1e-20))
        gbest = p[:, 0:1]
        gidx = jnp.zeros((_B, 1), jnp.int32)
        for c in range(1, _K):
            hit = p[:, c:c + 1] > gbest
            gidx = jnp.where(hit, c, gidx)
            gbest = jnp.where(hit, p[:, c:c + 1], gbest)
        gr_ref[:, v:v + 1] = gidx


def _ddpm_kernel(t_ref, x_ref, lp_ref, gr_ref, out_ref):
    i = pl.program_id(0)

    si = jax.lax.broadcasted_iota(jnp.int32, (8, _CW), 0)
    qi = jax.lax.broadcasted_iota(jnp.int32, (8, _CW), 1)
    pattern = (4 * qi + (si & 3) + (si >> 2) * (2 * _TL)).astype(jnp.uint32)

    for r in range(_RP):
        t0 = t_ref[i * _RP + r]

        def lp(v, c, r=r):
            return lp_ref[0, r:r + 1, 4 * v + c:4 * v + c + 1]

        def gr(v, r=r):
            return gr_ref[0, r:r + 1, v:v + 1]

        base = (i * _RP + r) * (_L * _K) + 42
        xv = x_ref[0, r:r + 1, :]
        for k in range(_NC // _CW):
            ctr = pattern + jnp.uint32(base + 4 * k * _CW)
            bits = _threefry_zero_hi(ctr)
            fb = jax.lax.bitcast_convert_type(
                (bits >> np.uint32(9)) | np.uint32(0x3F800000), jnp.float32)
            u = (fb - np.float32(1.0)) + _TINY
            g = -jnp.log(-jnp.log(u))
            for half in range(2):
                off = half * _NC + k * _CW
                v = xv[:, off:off + _CW]
                r0 = 4 * half
                best = g[r0:r0 + 1, :] + _sel4(
                    v, lp(0, 0), lp(1, 0), lp(2, 0), lp(3, 0))
                idx = jnp.zeros((1, _CW), jnp.int32)
                for c in range(1, _K):
                    sc = g[r0 + c:r0 + c + 1, :] + _sel4(
                        v, lp(0, c), lp(1, c), lp(2, c), lp(3, c))
                    hit = sc > best
                    idx = jnp.where(hit, c, idx)
                    best = jnp.where(hit, sc, best)
                grd = _sel4(v, gr(0), gr(1), gr(2), gr(3))
                res = jnp.where(t0 == 0, grd, idx)
                out_ref[0, r:r + 1, pl.ds(off, _CW)] = res


def kernel(x_t, t, emb, temb, W, b):
    x3 = x_t.reshape(_B // _RP, _RP, _L)
    b2 = b.reshape(1, _K)
    t2 = t.reshape(_B, 1)

    lp_all, gr_all = pl.pallas_call(
        _table_kernel,
        in_specs=[
            pl.BlockSpec((_B, 1), lambda: (0, 0)),
            pl.BlockSpec((_K, _D), lambda: (0, 0)),
            pl.BlockSpec((_T, _D), lambda: (0, 0)),
            pl.BlockSpec((_D, _K), lambda: (0, 0)),
            pl.BlockSpec((1, _K), lambda: (0, 0)),
        ],
        out_specs=[
            pl.BlockSpec((_B, 4 * _K), lambda: (0, 0)),
            pl.BlockSpec((_B, _K), lambda: (0, 0)),
        ],
        out_shape=[
            jax.ShapeDtypeStruct((_B, 4 * _K), jnp.float32),
            jax.ShapeDtypeStruct((_B, _K), jnp.int32),
        ],
    )(t2, emb, temb, W, b2)

    out = pl.pallas_call(
        _ddpm_kernel,
        grid_spec=pltpu.PrefetchScalarGridSpec(
            num_scalar_prefetch=1,
            grid=(_B // _RP,),
            in_specs=[
                pl.BlockSpec((1, _RP, _TL), lambda i, t_s: (i, 0, 0)),
                pl.BlockSpec((1, _RP, 4 * _K), lambda i, t_s: (i, 0, 0)),
                pl.BlockSpec((1, _RP, _K), lambda i, t_s: (i, 0, 0)),
            ],
            out_specs=pl.BlockSpec((1, _RP, _TL), lambda i, t_s: (i, 0, 0)),
        ),
        out_shape=jax.ShapeDtypeStruct((_B // _RP, _RP, _L), jnp.int32),
        compiler_params=pltpu.CompilerParams(
            dimension_semantics=("parallel",)),
    )(t, x3, lp_all.reshape(_B // _RP, _RP, 4 * _K),
      gr_all.reshape(_B // _RP, _RP, _K))
    return out.reshape(_B, _L)


# speedup vs baseline: 40.7396x; 1.0650x over previous
import numpy as np
import jax
import jax.numpy as jnp
from jax.experimental import pallas as pl
from jax.experimental.pallas import tpu as pltpu

_B, _L, _K, _D, _T = 128, 8192, 4, 256, 100
_RP = 4
_TL = 8192
_NC = _TL // 2
_CW = 1024

_KS0 = np.uint32(0)
_KS1 = np.uint32(42)
_KS2 = np.uint32(0x1BD11BDA) ^ _KS0 ^ _KS1
_ROT = ((13, 15, 26, 6), (17, 29, 16, 24))
_TINY = np.float32(np.finfo(np.float32).tiny)


def _rotl(x, r):
    return (x << np.uint32(r)) | (x >> np.uint32(32 - r))


def _threefry_zero_hi(x1):
    ks = (_KS0, _KS1, _KS2)
    x0 = x1
    x1 = _rotl(x1, 13)
    x1 = x0 ^ x1
    for r in (15, 26, 6):
        x0 = x0 + x1
        x1 = _rotl(x1, r)
        x1 = x0 ^ x1
    x0 = x0 + ks[1]
    x1 = x1 + ks[2] + np.uint32(1)
    for i in range(1, 5):
        for r in _ROT[i % 2]:
            x0 = x0 + x1
            x1 = _rotl(x1, r)
            x1 = x0 ^ x1
        x0 = x0 + ks[(i + 1) % 3]
        x1 = x1 + ks[(i + 2) % 3] + np.uint32(i + 1)
    return x0 ^ x1


def _sel4(v, a0, a1, a2, a3):
    return jnp.where(v == 0, a0, jnp.where(v == 1, a1, jnp.where(v == 2, a2, a3)))


def _table_kernel(t2_ref, emb_ref, temb_ref, w_ref, b_ref, lp_ref):
    t2 = t2_ref[...]
    oh = (jax.lax.broadcasted_iota(jnp.int32, (_B, _T), 1) == t2)
    h_all = jax.lax.dot_general(oh.astype(jnp.float32), temb_ref[...],
                                (((1,), (0,)), ((), ())),
                                precision=jax.lax.Precision.HIGHEST,
                                preferred_element_type=jnp.float32)
    for v in range(_K):
        hv = jax.nn.gelu(h_all + emb_ref[v:v + 1, :])
        lg = jax.lax.dot_general(hv, w_ref[...], (((1,), (0,)), ((), ())),
                                 precision=jax.lax.Precision.HIGHEST,
                                 preferred_element_type=jnp.float32)
        lg = lg + b_ref[...]
        m = jnp.maximum(jnp.maximum(lg[:, 0:1], lg[:, 1:2]),
                        jnp.maximum(lg[:, 2:3], lg[:, 3:4]))
        e = jnp.exp(lg - m)
        p = e / (e[:, 0:1] + e[:, 1:2] + e[:, 2:3] + e[:, 3:4])
        lpv = jnp.log(p + np.float32(1e-20))
        gbest = p[:, 0:1]
        gidx = jnp.zeros((_B, 1), jnp.int32)
        for c in range(1, _K):
            hit = p[:, c:c + 1] > gbest
            gidx = jnp.where(hit, c, gidx)
            gbest = jnp.where(hit, p[:, c:c + 1], gbest)
        ci = jax.lax.broadcasted_iota(jnp.int32, (_B, _K), 1)
        det = jnp.where(ci == gidx, np.float32(0.0), np.float32(-1e30))
        lp_ref[:, 4 * v:4 * v + 4] = jnp.where(t2 == 0, det, lpv)


def _ddpm_kernel(x_ref, lp_ref, out_ref):
    i = pl.program_id(0)

    si = jax.lax.broadcasted_iota(jnp.int32, (8, _CW), 0)
    qi = jax.lax.broadcasted_iota(jnp.int32, (8, _CW), 1)
    pattern = (4 * qi + (si & 3) + (si >> 2) * (2 * _TL)).astype(jnp.uint32)

    for r in range(_RP):
        lpb = [[jnp.broadcast_to(
                    lp_ref[0, r:r + 1, 4 * v + c:4 * v + c + 1], (1, _CW))
                for c in range(_K)] for v in range(_K)]

        base = (i * _RP + r) * (_L * _K) + 42
        xv = x_ref[0, r:r + 1, :]
        for k in range(_NC // _CW):
            ctr = pattern + jnp.uint32(base + 4 * k * _CW)
            bits = _threefry_zero_hi(ctr)
            fb = jax.lax.bitcast_convert_type(
                (bits >> np.uint32(9)) | np.uint32(0x3F800000), jnp.float32)
            u = (fb - np.float32(1.0)) + _TINY
            g = -jnp.log(-jnp.log(u))
            for half in range(2):
                off = half * _NC + k * _CW
                v = xv[:, off:off + _CW]
                m0 = v == 0
                m1 = v == 1
                m2 = v == 2

                def sel(c, m0=m0, m1=m1, m2=m2):
                    return jnp.where(m0, lpb[0][c], jnp.where(
                        m1, lpb[1][c], jnp.where(m2, lpb[2][c], lpb[3][c])))

                r0 = 4 * half
                best = g[r0:r0 + 1, :] + sel(0)
                idx = jnp.zeros((1, _CW), jnp.int32)
                for c in range(1, _K):
                    sc = g[r0 + c:r0 + c + 1, :] + sel(c)
                    hit = sc > best
                    idx = jnp.where(hit, c, idx)
                    best = jnp.where(hit, sc, best)
                out_ref[0, r:r + 1, pl.ds(off, _CW)] = idx


def kernel(x_t, t, emb, temb, W, b):
    x3 = x_t.reshape(_B // _RP, _RP, _L)
    b2 = b.reshape(1, _K)
    t2 = t.reshape(_B, 1)

    lp_all = pl.pallas_call(
        _table_kernel,
        in_specs=[
            pl.BlockSpec((_B, 1), lambda: (0, 0)),
            pl.BlockSpec((_K, _D), lambda: (0, 0)),
            pl.BlockSpec((_T, _D), lambda: (0, 0)),
            pl.BlockSpec((_D, _K), lambda: (0, 0)),
            pl.BlockSpec((1, _K), lambda: (0, 0)),
        ],
        out_specs=pl.BlockSpec((_B, 4 * _K), lambda: (0, 0)),
        out_shape=jax.ShapeDtypeStruct((_B, 4 * _K), jnp.float32),
    )(t2, emb, temb, W, b2)

    out = pl.pallas_call(
        _ddpm_kernel,
        grid=(_B // _RP,),
        in_specs=[
            pl.BlockSpec((1, _RP, _TL), lambda i: (i, 0, 0)),
            pl.BlockSpec((1, _RP, 4 * _K), lambda i: (i, 0, 0)),
        ],
        out_specs=pl.BlockSpec((1, _RP, _TL), lambda i: (i, 0, 0)),
        out_shape=jax.ShapeDtypeStruct((_B // _RP, _RP, _L), jnp.int32),
        compiler_params=pltpu.CompilerParams(
            dimension_semantics=("parallel",)),
    )(x3, lp_all.reshape(_B // _RP, _RP, 4 * _K))
    return out.reshape(_B, _L)


# speedup vs baseline: 45.7041x; 1.1219x over previous
import numpy as np
import jax
import jax.numpy as jnp
from jax.experimental import pallas as pl
from jax.experimental.pallas import tpu as pltpu

_B, _L, _K, _D, _T = 128, 8192, 4, 256, 100
_RP = 8
_TL = 8192
_NC = _TL // 2
_CW = 1024

_KS0 = np.uint32(0)
_KS1 = np.uint32(42)
_KS2 = np.uint32(0x1BD11BDA) ^ _KS0 ^ _KS1
_ROT = ((13, 15, 26, 6), (17, 29, 16, 24))
_TINY = np.float32(np.finfo(np.float32).tiny)


def _rotl(x, r):
    return (x << np.uint32(r)) | (x >> np.uint32(32 - r))


def _threefry_zero_hi(x1):
    ks = (_KS0, _KS1, _KS2)
    x0 = x1
    x1 = _rotl(x1, 13)
    x1 = x0 ^ x1
    for r in (15, 26, 6):
        x0 = x0 + x1
        x1 = _rotl(x1, r)
        x1 = x0 ^ x1
    x0 = x0 + ks[1]
    x1 = x1 + ks[2] + np.uint32(1)
    for i in range(1, 5):
        for r in _ROT[i % 2]:
            x0 = x0 + x1
            x1 = _rotl(x1, r)
            x1 = x0 ^ x1
        x0 = x0 + ks[(i + 1) % 3]
        x1 = x1 + ks[(i + 2) % 3] + np.uint32(i + 1)
    return x0 ^ x1


def _sel4(v, a0, a1, a2, a3):
    return jnp.where(v == 0, a0, jnp.where(v == 1, a1, jnp.where(v == 2, a2, a3)))


def _table_body(t2_ref, emb_ref, temb_ref, w_ref, b_ref, lp_ref):
    t2 = t2_ref[...]
    oh = (jax.lax.broadcasted_iota(jnp.int32, (_B, _T), 1) == t2)
    h_all = jax.lax.dot_general(oh.astype(jnp.float32), temb_ref[...],
                                (((1,), (0,)), ((), ())),
                                precision=jax.lax.Precision.HIGHEST,
                                preferred_element_type=jnp.float32)
    for v in range(_K):
        hv = jax.nn.gelu(h_all + emb_ref[v:v + 1, :])
        lg = jax.lax.dot_general(hv, w_ref[...], (((1,), (0,)), ((), ())),
                                 precision=jax.lax.Precision.HIGHEST,
                                 preferred_element_type=jnp.float32)
        lg = lg + b_ref[...]
        m = jnp.maximum(jnp.maximum(lg[:, 0:1], lg[:, 1:2]),
                        jnp.maximum(lg[:, 2:3], lg[:, 3:4]))
        e = jnp.exp(lg - m)
        p = e / (e[:, 0:1] + e[:, 1:2] + e[:, 2:3] + e[:, 3:4])
        lpv = jnp.log(p + np.float32(1e-20))
        gbest = p[:, 0:1]
        gidx = jnp.zeros((_B, 1), jnp.int32)
        for c in range(1, _K):
            hit = p[:, c:c + 1] > gbest
            gidx = jnp.where(hit, c, gidx)
            gbest = jnp.where(hit, p[:, c:c + 1], gbest)
        ci = jax.lax.broadcasted_iota(jnp.int32, (_B, _K), 1)
        det = jnp.where(ci == gidx, np.float32(0.0), np.float32(-1e30))
        lp_ref[:, 4 * v:4 * v + 4] = jnp.where(t2 == 0, det, lpv)


def _ddpm_kernel(x_ref, t2_ref, emb_ref, temb_ref, w_ref, b_ref, out_ref,
                 lp_ref):
    i = pl.program_id(0)

    @pl.when(i == 0)
    def _():
        _table_body(t2_ref, emb_ref, temb_ref, w_ref, b_ref, lp_ref)

    si = jax.lax.broadcasted_iota(jnp.int32, (8, _CW), 0)
    qi = jax.lax.broadcasted_iota(jnp.int32, (8, _CW), 1)
    pattern = (4 * qi + (si & 3) + (si >> 2) * (2 * _TL)).astype(jnp.uint32)

    for r in range(_RP):
        row = i * _RP + r
        lpb = [[jnp.broadcast_to(
                    lp_ref[pl.ds(row, 1), 4 * v + c:4 * v + c + 1], (1, _CW))
                for c in range(_K)] for v in range(_K)]

        base = (i * _RP + r) * (_L * _K) + 42
        xv = x_ref[0, r:r + 1, :]
        for k in range(_NC // _CW):
            ctr = pattern + jnp.uint32(base + 4 * k * _CW)
            bits = _threefry_zero_hi(ctr)
            fb = jax.lax.bitcast_convert_type(
                (bits >> np.uint32(9)) | np.uint32(0x3F800000), jnp.float32)
            u = (fb - np.float32(1.0)) + _TINY
            g = -jnp.log(-jnp.log(u))
            for half in range(2):
                off = half * _NC + k * _CW
                v = xv[:, off:off + _CW]
                m0 = v == 0
                m1 = v == 1
                m2 = v == 2

                def sel(c, m0=m0, m1=m1, m2=m2):
                    return jnp.where(m0, lpb[0][c], jnp.where(
                        m1, lpb[1][c], jnp.where(m2, lpb[2][c], lpb[3][c])))

                r0 = 4 * half
                best = g[r0:r0 + 1, :] + sel(0)
                idx = jnp.zeros((1, _CW), jnp.int32)
                for c in range(1, _K):
                    sc = g[r0 + c:r0 + c + 1, :] + sel(c)
                    hit = sc > best
                    idx = jnp.where(hit, c, idx)
                    best = jnp.where(hit, sc, best)
                out_ref[0, r:r + 1, pl.ds(off, _CW)] = idx


def kernel(x_t, t, emb, temb, W, b):
    x3 = x_t.reshape(_B // _RP, _RP, _L)
    b2 = b.reshape(1, _K)
    t2 = t.reshape(_B, 1)

    out = pl.pallas_call(
        _ddpm_kernel,
        grid=(_B // _RP,),
        in_specs=[
            pl.BlockSpec((1, _RP, _TL), lambda i: (i, 0, 0)),
            pl.BlockSpec((_B, 1), lambda i: (0, 0)),
            pl.BlockSpec((_K, _D), lambda i: (0, 0)),
            pl.BlockSpec((_T, _D), lambda i: (0, 0)),
            pl.BlockSpec((_D, _K), lambda i: (0, 0)),
            pl.BlockSpec((1, _K), lambda i: (0, 0)),
        ],
        out_specs=pl.BlockSpec((1, _RP, _TL), lambda i: (i, 0, 0)),
        out_shape=jax.ShapeDtypeStruct((_B // _RP, _RP, _L), jnp.int32),
        scratch_shapes=[pltpu.VMEM((_B, 4 * _K), jnp.float32)],
        compiler_params=pltpu.CompilerParams(
            dimension_semantics=("arbitrary",)),
    )(x3, t2, emb, temb, W, b2)
    return out.reshape(_B, _L)


# speedup vs baseline: 45.8450x; 1.0031x over previous
import numpy as np
import jax
import jax.numpy as jnp
from jax.experimental import pallas as pl
from jax.experimental.pallas import tpu as pltpu

_B, _L, _K, _D, _T = 128, 8192, 4, 256, 100
_RP = 16
_TL = 8192
_NC = _TL // 2
_CW = 1024

_KS0 = np.uint32(0)
_KS1 = np.uint32(42)
_KS2 = np.uint32(0x1BD11BDA) ^ _KS0 ^ _KS1
_ROT = ((13, 15, 26, 6), (17, 29, 16, 24))
_TINY = np.float32(np.finfo(np.float32).tiny)


def _rotl(x, r):
    return (x << np.uint32(r)) | (x >> np.uint32(32 - r))


def _threefry_zero_hi(x1):
    ks = (_KS0, _KS1, _KS2)
    x0 = x1
    x1 = _rotl(x1, 13)
    x1 = x0 ^ x1
    for r in (15, 26, 6):
        x0 = x0 + x1
        x1 = _rotl(x1, r)
        x1 = x0 ^ x1
    x0 = x0 + ks[1]
    x1 = x1 + ks[2] + np.uint32(1)
    for i in range(1, 5):
        for r in _ROT[i % 2]:
            x0 = x0 + x1
            x1 = _rotl(x1, r)
            x1 = x0 ^ x1
        x0 = x0 + ks[(i + 1) % 3]
        x1 = x1 + ks[(i + 2) % 3] + np.uint32(i + 1)
    return x0 ^ x1


def _sel4(v, a0, a1, a2, a3):
    return jnp.where(v == 0, a0, jnp.where(v == 1, a1, jnp.where(v == 2, a2, a3)))


def _table_body(t2_ref, emb_ref, temb_ref, w_ref, b_ref, lp_ref):
    t2 = t2_ref[...]
    oh = (jax.lax.broadcasted_iota(jnp.int32, (_B, _T), 1) == t2)
    h_all = jax.lax.dot_general(oh.astype(jnp.float32), temb_ref[...],
                                (((1,), (0,)), ((), ())),
                                precision=jax.lax.Precision.HIGHEST,
                                preferred_element_type=jnp.float32)
    for v in range(_K):
        hv = jax.nn.gelu(h_all + emb_ref[v:v + 1, :])
        lg = jax.lax.dot_general(hv, w_ref[...], (((1,), (0,)), ((), ())),
                                 precision=jax.lax.Precision.HIGHEST,
                                 preferred_element_type=jnp.float32)
        lg = lg + b_ref[...]
        m = jnp.maximum(jnp.maximum(lg[:, 0:1], lg[:, 1:2]),
                        jnp.maximum(lg[:, 2:3], lg[:, 3:4]))
        e = jnp.exp(lg - m)
        p = e / (e[:, 0:1] + e[:, 1:2] + e[:, 2:3] + e[:, 3:4])
        lpv = jnp.log(p + np.float32(1e-20))
        gbest = p[:, 0:1]
        gidx = jnp.zeros((_B, 1), jnp.int32)
        for c in range(1, _K):
            hit = p[:, c:c + 1] > gbest
            gidx = jnp.where(hit, c, gidx)
            gbest = jnp.where(hit, p[:, c:c + 1], gbest)
        ci = jax.lax.broadcasted_iota(jnp.int32, (_B, _K), 1)
        det = jnp.where(ci == gidx, np.float32(0.0), np.float32(-1e30))
        lp_ref[:, 4 * v:4 * v + 4] = jnp.where(t2 == 0, det, lpv)


def _ddpm_kernel(x_ref, t2_ref, emb_ref, temb_ref, w_ref, b_ref, out_ref,
                 lp_ref):
    i = pl.program_id(0)

    @pl.when(i == 0)
    def _():
        _table_body(t2_ref, emb_ref, temb_ref, w_ref, b_ref, lp_ref)

    si = jax.lax.broadcasted_iota(jnp.int32, (8, _CW), 0)
    qi = jax.lax.broadcasted_iota(jnp.int32, (8, _CW), 1)
    pattern = (4 * qi + (si & 3) + (si >> 2) * (2 * _TL)).astype(jnp.uint32)

    for r in range(_RP):
        row = i * _RP + r
        lprow = lp_ref[pl.ds(row, 1), :]
        lpb = [[jnp.broadcast_to(
                    lprow[:, 4 * v + c:4 * v + c + 1], (1, _CW))
                for c in range(_K)] for v in range(_K)]

        base = (i * _RP + r) * (_L * _K) + 42
        xv = x_ref[0, r:r + 1, :]
        for k in range(_NC // _CW):
            ctr = pattern + jnp.uint32(base + 4 * k * _CW)
            bits = _threefry_zero_hi(ctr)
            fb = jax.lax.bitcast_convert_type(
                (bits >> np.uint32(9)) | np.uint32(0x3F800000), jnp.float32)
            u = (fb - np.float32(1.0)) + _TINY
            g = -jnp.log(-jnp.log(u))
            for half in range(2):
                off = half * _NC + k * _CW
                v = xv[:, off:off + _CW]
                m0 = v == 0
                m1 = v == 1
                m2 = v == 2

                def sel(c, m0=m0, m1=m1, m2=m2):
                    return jnp.where(m0, lpb[0][c], jnp.where(
                        m1, lpb[1][c], jnp.where(m2, lpb[2][c], lpb[3][c])))

                r0 = 4 * half
                best = g[r0:r0 + 1, :] + sel(0)
                idx = jnp.zeros((1, _CW), jnp.int32)
                for c in range(1, _K):
                    sc = g[r0 + c:r0 + c + 1, :] + sel(c)
                    hit = sc > best
                    idx = jnp.where(hit, c, idx)
                    best = jnp.where(hit, sc, best)
                out_ref[0, r:r + 1, pl.ds(off, _CW)] = idx


def kernel(x_t, t, emb, temb, W, b):
    x3 = x_t.reshape(_B // _RP, _RP, _L)
    b2 = b.reshape(1, _K)
    t2 = t.reshape(_B, 1)

    out = pl.pallas_call(
        _ddpm_kernel,
        grid=(_B // _RP,),
        in_specs=[
            pl.BlockSpec((1, _RP, _TL), lambda i: (i, 0, 0)),
            pl.BlockSpec((_B, 1), lambda i: (0, 0)),
            pl.BlockSpec((_K, _D), lambda i: (0, 0)),
            pl.BlockSpec((_T, _D), lambda i: (0, 0)),
            pl.BlockSpec((_D, _K), lambda i: (0, 0)),
            pl.BlockSpec((1, _K), lambda i: (0, 0)),
        ],
        out_specs=pl.BlockSpec((1, _RP, _TL), lambda i: (i, 0, 0)),
        out_shape=jax.ShapeDtypeStruct((_B // _RP, _RP, _L), jnp.int32),
        scratch_shapes=[pltpu.VMEM((_B, 4 * _K), jnp.float32)],
        compiler_params=pltpu.CompilerParams(
            dimension_semantics=("arbitrary",)),
    )(x3, t2, emb, temb, W, b2)
    return out.reshape(_B, _L)


# speedup vs baseline: 53.5711x; 1.1685x over previous
import numpy as np
import jax
import jax.numpy as jnp
from jax.experimental import pallas as pl
from jax.experimental.pallas import tpu as pltpu

_B, _L, _K, _D, _T = 128, 8192, 4, 256, 100
_RP = 16
_TL = 8192
_NC = _TL // 2
_CW = 1024

_KS0 = np.uint32(0)
_KS1 = np.uint32(42)
_KS2 = np.uint32(0x1BD11BDA) ^ _KS0 ^ _KS1
_ROT = ((13, 15, 26, 6), (17, 29, 16, 24))
_TINY = np.float32(np.finfo(np.float32).tiny)


def _rotl(x, r):
    return (x << np.uint32(r)) | (x >> np.uint32(32 - r))


def _threefry_zero_hi(x1):
    ks = (_KS0, _KS1, _KS2)
    x0 = x1
    x1 = _rotl(x1, 13)
    x1 = x0 ^ x1
    for r in (15, 26, 6):
        x0 = x0 + x1
        x1 = _rotl(x1, r)
        x1 = x0 ^ x1
    x0 = x0 + ks[1]
    x1 = x1 + ks[2] + np.uint32(1)
    for i in range(1, 5):
        for r in _ROT[i % 2]:
            x0 = x0 + x1
            x1 = _rotl(x1, r)
            x1 = x0 ^ x1
        x0 = x0 + ks[(i + 1) % 3]
        x1 = x1 + ks[(i + 2) % 3] + np.uint32(i + 1)
    return x0 ^ x1


def _sel4(v, a0, a1, a2, a3):
    return jnp.where(v == 0, a0, jnp.where(v == 1, a1, jnp.where(v == 2, a2, a3)))


def _table_body(t2_ref, emb_ref, temb_ref, w_ref, b_ref, lp_ref):
    t2 = t2_ref[...]
    oh = (jax.lax.broadcasted_iota(jnp.int32, (_B, _T), 1) == t2)
    h_all = jax.lax.dot_general(oh.astype(jnp.float32), temb_ref[...],
                                (((1,), (0,)), ((), ())),
                                precision=jax.lax.Precision.HIGHEST,
                                preferred_element_type=jnp.float32)
    for v in range(_K):
        hv = jax.nn.gelu(h_all + emb_ref[v:v + 1, :])
        lg = jax.lax.dot_general(hv, w_ref[...], (((1,), (0,)), ((), ())),
                                 precision=jax.lax.Precision.HIGHEST,
                                 preferred_element_type=jnp.float32)
        lg = lg + b_ref[...]
        m = jnp.maximum(jnp.maximum(lg[:, 0:1], lg[:, 1:2]),
                        jnp.maximum(lg[:, 2:3], lg[:, 3:4]))
        e = jnp.exp(lg - m)
        p = e / (e[:, 0:1] + e[:, 1:2] + e[:, 2:3] + e[:, 3:4])
        lpv = jnp.log(p + np.float32(1e-20))
        gbest = p[:, 0:1]
        gidx = jnp.zeros((_B, 1), jnp.int32)
        for c in range(1, _K):
            hit = p[:, c:c + 1] > gbest
            gidx = jnp.where(hit, c, gidx)
            gbest = jnp.where(hit, p[:, c:c + 1], gbest)
        ci = jax.lax.broadcasted_iota(jnp.int32, (_B, _K), 1)
        det = jnp.where(ci == gidx, np.float32(0.0), np.float32(-1e30))
        lp_ref[:, 4 * v:4 * v + 4] = jnp.where(t2 == 0, det, lpv)


def _ddpm_kernel(x_ref, t2_ref, emb_ref, temb_ref, w_ref, b_ref, out_ref,
                 lp_ref):
    i = pl.program_id(0)

    @pl.when(i == 0)
    def _():
        _table_body(t2_ref, emb_ref, temb_ref, w_ref, b_ref, lp_ref)

    si = jax.lax.broadcasted_iota(jnp.int32, (8, _CW), 0)
    qi = jax.lax.broadcasted_iota(jnp.int32, (8, _CW), 1)
    pattern = (4 * qi + (si & 3) + (si >> 2) * (2 * _TL)).astype(jnp.uint32)
    idx0 = si & 3
    idxr1 = (si + 1) & 3
    shalf = si < 4

    for r in range(_RP):
        row = i * _RP + r
        lprow = lp_ref[pl.ds(row, 1), :]
        cm0 = idx0 == 0
        cm1 = idx0 == 1
        cm2 = idx0 == 2
        lv8 = [jnp.where(cm0, lprow[:, 4 * v:4 * v + 1],
               jnp.where(cm1, lprow[:, 4 * v + 1:4 * v + 2],
               jnp.where(cm2, lprow[:, 4 * v + 2:4 * v + 3],
                         lprow[:, 4 * v + 3:4 * v + 4])))
               for v in range(_K)]

        base = (i * _RP + r) * (_L * _K) + 42
        xv = x_ref[0, r:r + 1, :]
        for k in range(_NC // _CW):
            ctr = pattern + jnp.uint32(base + 4 * k * _CW)
            bits = _threefry_zero_hi(ctr)
            fb = jax.lax.bitcast_convert_type(
                (bits >> np.uint32(9)) | np.uint32(0x3F800000), jnp.float32)
            u = (fb - np.float32(1.0)) + _TINY
            g = -jnp.log(-jnp.log(u))

            vlo = jnp.broadcast_to(xv[:, k * _CW:(k + 1) * _CW], (8, _CW))
            vhi = jnp.broadcast_to(
                xv[:, _NC + k * _CW:_NC + (k + 1) * _CW], (8, _CW))
            v8 = jnp.where(shalf, vlo, vhi)
            score = g + jnp.where(v8 == 0, lv8[0],
                        jnp.where(v8 == 1, lv8[1],
                        jnp.where(v8 == 2, lv8[2], lv8[3])))

            s_r = jnp.roll(score, -1, axis=0)
            c1 = s_r > score
            s1 = jnp.where(c1, s_r, score)
            i1 = jnp.where(c1, idxr1, idx0)
            c2 = jnp.roll(s1, -2, axis=0) > s1
            i2 = jnp.where(c2, jnp.roll(i1, -2, axis=0), i1)
            out_ref[0, r:r + 1, pl.ds(k * _CW, _CW)] = i2[0:1, :]
            out_ref[0, r:r + 1, pl.ds(_NC + k * _CW, _CW)] = i2[4:5, :]


def kernel(x_t, t, emb, temb, W, b):
    x3 = x_t.reshape(_B // _RP, _RP, _L)
    b2 = b.reshape(1, _K)
    t2 = t.reshape(_B, 1)

    out = pl.pallas_call(
        _ddpm_kernel,
        grid=(_B // _RP,),
        in_specs=[
            pl.BlockSpec((1, _RP, _TL), lambda i: (i, 0, 0)),
            pl.BlockSpec((_B, 1), lambda i: (0, 0)),
            pl.BlockSpec((_K, _D), lambda i: (0, 0)),
            pl.BlockSpec((_T, _D), lambda i: (0, 0)),
            pl.BlockSpec((_D, _K), lambda i: (0, 0)),
            pl.BlockSpec((1, _K), lambda i: (0, 0)),
        ],
        out_specs=pl.BlockSpec((1, _RP, _TL), lambda i: (i, 0, 0)),
        out_shape=jax.ShapeDtypeStruct((_B // _RP, _RP, _L), jnp.int32),
        scratch_shapes=[pltpu.VMEM((_B, 4 * _K), jnp.float32)],
        compiler_params=pltpu.CompilerParams(
            dimension_semantics=("arbitrary",)),
    )(x3, t2, emb, temb, W, b2)
    return out.reshape(_B, _L)
